# Initial kernel scaffold; baseline (speedup 1.0000x reference)
import jax
import jax.numpy as jnp
from jax.experimental import pallas as pl


def kernel(token_ids, token_table, pos_table):
    raise NotImplementedError("write your pallas kernel here")


# speedup vs baseline: 1.3172x; 1.3172x over previous
import functools

import jax
import jax.numpy as jnp
from jax import lax
from jax.experimental import pallas as pl
from jax.experimental.pallas import tpu as pltpu
from jax.experimental.pallas import tpu_sc as plsc

NC = 2
NS = 16
NW = NC * NS


@functools.lru_cache(maxsize=None)
def _emb_kernel(seq, n_rows, d, chunk):
    rows_per_w = n_rows // NW
    nchunk = rows_per_w // chunk
    mesh = plsc.VectorSubcoreMesh(core_axis_name="c", subcore_axis_name="s")

    @functools.partial(
        pl.kernel,
        mesh=mesh,
        compiler_params=pltpu.CompilerParams(use_tc_tiling_on_sc=False),
        out_type=jax.ShapeDtypeStruct((n_rows, d), jnp.float32),
        scratch_types=[
            pltpu.VMEM((chunk,), jnp.int32),
            pltpu.VMEM((chunk, d), jnp.float32),
            pltpu.VMEM((chunk, d), jnp.float32),
            pltpu.SemaphoreType.DMA,
        ],
    )
    def k(ids_hbm, tok_hbm, pos_hbm, out_hbm, idx_v, rows_v, pos2_v, sem):
        wid = lax.axis_index("s") * NC + lax.axis_index("c")
        base = wid * rows_per_w
        for r in range(chunk // seq):
            pltpu.sync_copy(pos_hbm.at[pl.ds(0, seq)],
                            pos2_v.at[pl.ds(r * seq, seq)])

        def chunk_body(c, carry):
            row0 = base + c * chunk
            pltpu.sync_copy(ids_hbm.at[pl.ds(row0, chunk)], idx_v)
            pltpu.async_copy(tok_hbm.at[idx_v], rows_v, sem).wait()

            U = 8

            def add_body(jo, carry2):
                for u in range(U):
                    j = jo * U + u
                    for h in range(d // 16):
                        sl = pl.ds(h * 16, 16)
                        rows_v[j, sl] = rows_v[j, sl] + pos2_v[j, sl]
                return carry2

            lax.fori_loop(0, chunk // U, add_body, 0)
            pltpu.sync_copy(rows_v, out_hbm.at[pl.ds(row0, chunk)])
            return carry

        lax.fori_loop(0, nchunk, chunk_body, 0)

    return k


def kernel(token_ids, token_table, pos_table):
    b, seq = token_ids.shape
    d = token_table.shape[1]
    n_rows = b * seq
    chunk = 2 * seq
    ids_flat = token_ids.reshape(n_rows).astype(jnp.int32)
    out = _emb_kernel(seq, n_rows, d, chunk)(ids_flat, token_table, pos_table)
    return out.reshape(b, seq, d)


# speedup vs baseline: 1.4506x; 1.1013x over previous
import functools

import jax
import jax.numpy as jnp
from jax import lax
from jax.experimental import pallas as pl
from jax.experimental.pallas import tpu as pltpu
from jax.experimental.pallas import tpu_sc as plsc

NC = 2
NS = 16
NW = NC * NS
NBUF = 3


@functools.lru_cache(maxsize=None)
def _emb_kernel(seq, n_rows, d, chunk):
    rows_per_w = n_rows // NW
    nchunk = rows_per_w // chunk
    mesh = plsc.VectorSubcoreMesh(core_axis_name="c", subcore_axis_name="s")

    @functools.partial(
        pl.kernel,
        mesh=mesh,
        compiler_params=pltpu.CompilerParams(use_tc_tiling_on_sc=False),
        out_type=jax.ShapeDtypeStruct((n_rows, d), jnp.float32),
        scratch_types=[
            pltpu.VMEM((NBUF, chunk), jnp.int32),
            pltpu.VMEM((NBUF, chunk, d), jnp.float32),
            pltpu.VMEM((chunk, d), jnp.float32),
            [pltpu.SemaphoreType.DMA] * NBUF,
            [pltpu.SemaphoreType.DMA] * NBUF,
            [pltpu.SemaphoreType.DMA] * NBUF,
        ],
    )
    def k(ids_hbm, tok_hbm, pos_hbm, out_hbm, idx_v, rows_v, pos2_v,
          sem_i, sem_g, sem_o):
        wid = lax.axis_index("s") * NC + lax.axis_index("c")
        base = wid * rows_per_w
        for r in range(chunk // seq):
            pltpu.sync_copy(pos_hbm.at[pl.ds(0, seq)],
                            pos2_v.at[pl.ds(r * seq, seq)])

        def start_idx(c, s):
            return pltpu.async_copy(
                ids_hbm.at[pl.ds(base + c * chunk, chunk)],
                idx_v.at[s], sem_i[s])

        def start_gather(s):
            return pltpu.async_copy(
                tok_hbm.at[idx_v.at[s]], rows_v.at[s], sem_g[s])

        def start_out(c, s):
            return pltpu.async_copy(
                rows_v.at[s], out_hbm.at[pl.ds(base + c * chunk, chunk)],
                sem_o[s])

        def add_pos(s):
            U = 8

            def add_body(jo, carry):
                for u in range(U):
                    j = jo * U + u
                    for h in range(d // 16):
                        sl = pl.ds(h * 16, 16)
                        rows_v[s, j, sl] = rows_v[s, j, sl] + pos2_v[j, sl]
                return carry

            lax.fori_loop(0, chunk // U, add_body, 0)

        h_i, h_g, h_o = {}, {}, {}
        h_i[0] = start_idx(0, 0)
        if nchunk > 1:
            h_i[1] = start_idx(1, 1)
        h_i[0].wait()
        h_g[0] = start_gather(0)

        for c in range(nchunk):
            s0, s1, s2 = c % NBUF, (c + 1) % NBUF, (c + 2) % NBUF
            if c + 2 < nchunk:
                if c + 2 >= NBUF:
                    h_o[c + 2 - NBUF].wait()
                h_i[c + 2] = start_idx(c + 2, s2)
            if c + 1 < nchunk:
                h_i[c + 1].wait()
                h_g[c + 1] = start_gather(s1)
            h_g[c].wait()
            add_pos(s0)
            h_o[c] = start_out(c, s0)

        for c in range(max(0, nchunk - NBUF), nchunk):
            h_o[c].wait()

    return k


def kernel(token_ids, token_table, pos_table):
    b, seq = token_ids.shape
    d = token_table.shape[1]
    n_rows = b * seq
    chunk = 2 * seq
    ids_flat = token_ids.reshape(n_rows).astype(jnp.int32)
    out = _emb_kernel(seq, n_rows, d, chunk)(ids_flat, token_table, pos_table)
    return out.reshape(b, seq, d)
